# baseline (speedup 1.0000x reference)
import jax
import jax.numpy as jnp
from jax import lax
from jax.experimental import pallas as pl
from jax.experimental.pallas import tpu as pltpu
from jax.experimental.pallas import tpu_sc as plsc

L = 16
D = 128
B = 4096
SEQ = 50
NW = 32
B_W = B // NW
NG = D // L
NBG = B_W // L


def _body(s_hbm, w_hbm, e_hbm, out_hbm,
          w_v, e_v, s_v, buf0, buf1, buf2, semo0, semo1, semo2):
    wid = lax.axis_index("s") * 2 + lax.axis_index("c")
    bbase = wid * B_W

    pltpu.sync_copy(s_hbm.at[:, pl.ds(bbase, B_W)], s_v)
    pltpu.sync_copy(w_hbm, w_v)
    pltpu.sync_copy(e_hbm, e_v)

    wg = [w_v[pl.ds(g * L, L)] for g in range(NG)]
    e0g = [e_v[0, pl.ds(g * L, L)] for g in range(NG)]
    e1g = [e_v[1, pl.ds(g * L, L)] for g in range(NG)]

    bufs = (buf0, buf1, buf2)
    semos = (semo0, semo1, semo2)

    def _chunk(c, b):
        buf = bufs[b]
        semo = semos[b]

        @pl.when(c >= 3)
        def _wait_prev_out():
            pltpu.make_async_copy(
                buf, out_hbm.at[c - 3, pl.ds(bbase, B_W)], semo
            ).wait()

        @pl.loop(0, NBG)
        def _grp(g):
            sv = s_v[c, pl.ds(g * L, L)]
            for j in range(L):
                sb = jnp.full((L,), sv[j], jnp.float32)
                nanb = sb != sb
                r = g * L + j
                for d in range(NG):
                    buf[r, pl.ds(d * L, L)] = jnp.where(
                        nanb, e1g[d], sb * wg[d] + e0g[d]
                    )

        pltpu.make_async_copy(
            buf, out_hbm.at[c, pl.ds(bbase, B_W)], semo
        ).start()

    @pl.loop(0, (SEQ - 2) // 3)
    def _outer(i):
        for b in range(3):
            _chunk(3 * i + b, b)

    for cc in (SEQ - 2, SEQ - 1):
        _chunk(cc, cc % 3)

    for cc in (SEQ - 3, SEQ - 2, SEQ - 1):
        pltpu.make_async_copy(
            bufs[cc % 3], out_hbm.at[cc, pl.ds(bbase, B_W)], semos[cc % 3]
        ).wait()


@jax.jit
def kernel(scalar, W_fc, emb_nan):
    s_t = jnp.transpose(scalar.reshape(B, SEQ), (1, 0))
    w_flat = W_fc.reshape(D)

    run = pl.kernel(
        _body,
        out_type=jax.ShapeDtypeStruct((SEQ, B, D), jnp.float32),
        mesh=plsc.VectorSubcoreMesh(core_axis_name="c", subcore_axis_name="s"),
        scratch_types=[
            pltpu.VMEM((D,), jnp.float32),
            pltpu.VMEM((2, D), jnp.float32),
            pltpu.VMEM((SEQ, B_W), jnp.float32),
            pltpu.VMEM((B_W, D), jnp.float32),
            pltpu.VMEM((B_W, D), jnp.float32),
            pltpu.VMEM((B_W, D), jnp.float32),
            pltpu.SemaphoreType.DMA,
            pltpu.SemaphoreType.DMA,
            pltpu.SemaphoreType.DMA,
        ],
        compiler_params=pltpu.CompilerParams(needs_layout_passes=False),
    )
    out_t = run(s_t, w_flat, emb_nan)
    return jnp.transpose(out_t, (1, 0, 2))


# speedup vs baseline: 1.0535x; 1.0535x over previous
import jax
import jax.numpy as jnp
from jax import lax
from jax.experimental import pallas as pl
from jax.experimental.pallas import tpu as pltpu
from jax.experimental.pallas import tpu_sc as plsc

L = 16
D = 128
B = 4096
SEQ = 50
NW = 32
B_W = B // NW
NG = D // L
NBG = B_W // L


def _body(s_hbm, w_hbm, e_hbm, out_hbm,
          w_v, e_v, s_v, buf0, buf1, semo0, semo1):
    wid = lax.axis_index("s") * 2 + lax.axis_index("c")
    bbase = wid * B_W

    pltpu.sync_copy(s_hbm.at[:, pl.ds(bbase, B_W)], s_v)
    pltpu.sync_copy(w_hbm, w_v)
    pltpu.sync_copy(e_hbm, e_v)

    wg = [w_v[pl.ds(g * L, L)] for g in range(NG)]
    e0g = [e_v[0, pl.ds(g * L, L)] for g in range(NG)]
    e1g = [e_v[1, pl.ds(g * L, L)] for g in range(NG)]

    bufs = (buf0, buf1)
    semos = (semo0, semo1)

    @pl.loop(0, SEQ // 2)
    def _outer(i):
        for b in range(2):
            c = 2 * i + b
            buf = bufs[b]
            semo = semos[b]

            @pl.when(c >= 2)
            def _wait_prev_out():
                pltpu.make_async_copy(
                    buf, out_hbm.at[c - 2, pl.ds(bbase, B_W)], semo
                ).wait()

            @pl.loop(0, NBG)
            def _grp(g):
                sv = s_v[c, pl.ds(g * L, L)]
                for j in range(L):
                    sb = jnp.full((L,), sv[j], jnp.float32)
                    nanb = sb != sb
                    r = g * L + j
                    for d in range(NG):
                        buf[r, pl.ds(d * L, L)] = jnp.where(
                            nanb, e1g[d], sb * wg[d] + e0g[d]
                        )

            pltpu.make_async_copy(
                buf, out_hbm.at[c, pl.ds(bbase, B_W)], semo
            ).start()

    pltpu.make_async_copy(
        buf0, out_hbm.at[SEQ - 2, pl.ds(bbase, B_W)], semo0
    ).wait()
    pltpu.make_async_copy(
        buf1, out_hbm.at[SEQ - 1, pl.ds(bbase, B_W)], semo1
    ).wait()


@jax.jit
def kernel(scalar, W_fc, emb_nan):
    s_t = jnp.transpose(scalar.reshape(B, SEQ), (1, 0))
    w_flat = W_fc.reshape(D)

    run = pl.kernel(
        _body,
        out_type=jax.ShapeDtypeStruct((SEQ, B, D), jnp.float32),
        mesh=plsc.VectorSubcoreMesh(core_axis_name="c", subcore_axis_name="s"),
        scratch_types=[
            pltpu.VMEM((D,), jnp.float32),
            pltpu.VMEM((2, D), jnp.float32),
            pltpu.VMEM((SEQ, B_W), jnp.float32),
            pltpu.VMEM((B_W, D), jnp.float32),
            pltpu.VMEM((B_W, D), jnp.float32),
            pltpu.SemaphoreType.DMA,
            pltpu.SemaphoreType.DMA,
        ],
        compiler_params=pltpu.CompilerParams(needs_layout_passes=False),
    )
    out_t = run(s_t, w_flat, emb_nan)
    return jnp.transpose(out_t, (1, 0, 2))
